# baseline (speedup 1.0000x reference)
import functools

import jax
import jax.numpy as jnp
from jax import lax
from jax.experimental import pallas as pl
from jax.experimental.pallas import tpu as pltpu
from jax.experimental.pallas import tpu_sc as plsc

_N = 10000
_E = 320000
_D = 128
_EDGE_D = 16
_G = 64
_L = 3
_OUT = 128

_NC = 2
_NS = 16
_NT = _NC * _NS
_B = 80
_CH = _E // (_NT * _B)
_IG = 25

_RPT = 624
_XB = 16
_LANES = 16


def _sc_agg_body(h_hbm, e_hbm, src_hbm, dst_hbm, out_hbm,
                 src_v, dst_v, rows_a, rows_b, msg_a, msg_b, agg_sh,
                 sga, sgb, sea, seb, ssa, ssb):
    c = lax.axis_index("c")
    s = lax.axis_index("s")
    tid = c * _NS + s

    zv = jnp.zeros((_LANES,), jnp.float32)

    def _zrow(r, carry):
        for j in range(_D // _LANES):
            rows_a[r, pl.ds(j * _LANES, _LANES)] = zv
        return carry

    lax.fori_loop(0, _B, _zrow, 0)
    start = s * _RPT
    for q in range(7):
        pltpu.sync_copy(rows_a, agg_sh.at[pl.ds(start + q * _B, _B)])

    @pl.when(s < _NS - 1)
    def _():
        pltpu.sync_copy(rows_a.at[pl.ds(0, 64)],
                        agg_sh.at[pl.ds(start + 7 * _B, 64)])

    @pl.when(s == _NS - 1)
    def _():
        pltpu.sync_copy(rows_a, agg_sh.at[pl.ds(start + 7 * _B, _B)])

    plsc.subcore_barrier()

    base = tid * (_CH * _B)

    def _eslice(g, k):
        return e_hbm.at[pl.ds(base + (g * _IG + k) * _B, _B)]

    def _gissue(g, k, rows_ref, sg):
        pltpu.async_copy(h_hbm.at[src_v.at[k]], rows_ref, sg)

    def _eissue(g, k, e_ref, se):
        pltpu.async_copy(_eslice(g, k), e_ref, se)

    def _swait(k, e_ref, ss):
        pltpu.make_async_copy(e_ref, agg_sh.at[dst_v.at[k]], ss).wait()

    def _comp(g, k, rows_ref, e_ref, sg, se, ss):
        pltpu.make_async_copy(h_hbm.at[src_v.at[k]], rows_ref, sg).wait()
        pltpu.make_async_copy(_eslice(g, k), e_ref, se).wait()

        def _row(r, rcarry):
            for j in range(_D // _LANES):
                sl = pl.ds(j * _LANES, _LANES)
                e_ref[r, sl] = jnp.maximum(e_ref[r, sl] + rows_ref[r, sl],
                                           0.0)
            return rcarry

        lax.fori_loop(0, _B, _row, 0)
        pltpu.async_copy(e_ref, agg_sh.at[dst_v.at[k]], ss, add=True)

    def _iblock(g, carry):
        pltpu.sync_copy(src_hbm.at[tid, g], src_v)
        pltpu.sync_copy(dst_hbm.at[tid, g], dst_v)
        _gissue(g, 0, rows_a, sga)
        _eissue(g, 0, msg_a, sea)
        _gissue(g, 1, rows_b, sgb)
        _eissue(g, 1, msg_b, seb)

        def _pair(m, kcarry):
            k0 = 2 * m
            _comp(g, k0, rows_a, msg_a, sga, sea, ssa)
            _gissue(g, k0 + 2, rows_a, sga)
            _comp(g, k0 + 1, rows_b, msg_b, sgb, seb, ssb)

            @pl.when(m < (_IG - 3) // 2)
            def _():
                _gissue(g, k0 + 3, rows_b, sgb)

            _swait(k0, msg_a, ssa)
            _eissue(g, k0 + 2, msg_a, sea)

            @pl.when(m < (_IG - 3) // 2)
            def _():
                _swait(k0 + 1, msg_b, ssb)
                _eissue(g, k0 + 3, msg_b, seb)

            return kcarry

        lax.fori_loop(0, (_IG - 1) // 2, _pair, 0)
        _comp(g, _IG - 1, rows_a, msg_a, sga, sea, ssa)
        _swait(_IG - 1, msg_a, ssa)
        _swait(_IG - 2, msg_b, ssb)
        return carry

    lax.fori_loop(0, _CH // _IG, _iblock, 0)
    plsc.subcore_barrier()

    for q in range(7):
        r0 = start + q * _B
        pltpu.sync_copy(agg_sh.at[pl.ds(r0, _B)], rows_a)
        pltpu.sync_copy(rows_a, out_hbm.at[c, pl.ds(r0, _B)])

    @pl.when(s < _NS - 1)
    def _():
        r0 = start + 7 * _B
        pltpu.sync_copy(agg_sh.at[pl.ds(r0, 64)], rows_a.at[pl.ds(0, 64)])
        pltpu.sync_copy(rows_a.at[pl.ds(0, 64)], out_hbm.at[c, pl.ds(r0, 64)])

    @pl.when(s == _NS - 1)
    def _():
        r0 = start + 7 * _B
        pltpu.sync_copy(agg_sh.at[pl.ds(r0, _B)], rows_a)
        pltpu.sync_copy(rows_a, out_hbm.at[c, pl.ds(r0, _B)])


_sc_agg = pl.kernel(
    _sc_agg_body,
    out_type=jax.ShapeDtypeStruct((_NC, _N, _D), jnp.float32),
    mesh=plsc.VectorSubcoreMesh(core_axis_name="c", subcore_axis_name="s"),
    scratch_types=[
        pltpu.VMEM((_IG, _B), jnp.int32),
        pltpu.VMEM((_IG, _B), jnp.int32),
        pltpu.VMEM((_B, _D), jnp.float32),
        pltpu.VMEM((_B, _D), jnp.float32),
        pltpu.VMEM((_B, _D), jnp.float32),
        pltpu.VMEM((_B, _D), jnp.float32),
        pltpu.VMEM_SHARED((_N, _D), jnp.float32),
        pltpu.SemaphoreType.DMA,
        pltpu.SemaphoreType.DMA,
        pltpu.SemaphoreType.DMA,
        pltpu.SemaphoreType.DMA,
        pltpu.SemaphoreType.DMA,
        pltpu.SemaphoreType.DMA,
    ],
)


_NB = 1000
_EB = 4000


def _enc_body(x_ref, w_ref, b_ref, o_ref):
    o_ref[...] = jnp.dot(x_ref[...], w_ref[...],
                         preferred_element_type=jnp.float32) + b_ref[...]


_enc_call = pl.pallas_call(
    _enc_body,
    grid=(_N // _NB,),
    in_specs=[
        pl.BlockSpec((_NB, _D), lambda i: (i, 0)),
        pl.BlockSpec((_D, _D), lambda i: (0, 0)),
        pl.BlockSpec((1, _D), lambda i: (0, 0)),
    ],
    out_specs=pl.BlockSpec((_NB, _D), lambda i: (i, 0)),
    out_shape=jax.ShapeDtypeStruct((_N, _D), jnp.float32),
)


def _edge_body(a_ref, w_ref, b_ref, o_ref):
    o_ref[...] = jnp.dot(a_ref[...], w_ref[...],
                         preferred_element_type=jnp.float32) + b_ref[...]


_edge_call = pl.pallas_call(
    _edge_body,
    grid=(_E // _EB,),
    in_specs=[
        pl.BlockSpec((_EB, _EDGE_D), lambda i: (i, 0)),
        pl.BlockSpec((_EDGE_D, _D), lambda i: (0, 0)),
        pl.BlockSpec((1, _D), lambda i: (0, 0)),
    ],
    out_specs=pl.BlockSpec((_EB, _D), lambda i: (i, 0)),
    out_shape=jax.ShapeDtypeStruct((_E, _D), jnp.float32),
)


def _mlp_body(h_ref, a_ref, s_ref, w1_ref, b1_ref, w2_ref, b2_ref, o_ref):
    scale = s_ref[0, 0]
    z = h_ref[...] * scale + a_ref[0] + a_ref[1]
    z = jnp.maximum(
        jnp.dot(z, w1_ref[...], preferred_element_type=jnp.float32)
        + b1_ref[...], 0.0)
    o_ref[...] = jnp.maximum(
        jnp.dot(z, w2_ref[...], preferred_element_type=jnp.float32)
        + b2_ref[...], 0.0)


_mlp_call = pl.pallas_call(
    _mlp_body,
    grid=(_N // _NB,),
    in_specs=[
        pl.BlockSpec((_NB, _D), lambda i: (i, 0)),
        pl.BlockSpec((_NC, _NB, _D), lambda i: (0, i, 0)),
        pl.BlockSpec((1, 1), lambda i: (0, 0)),
        pl.BlockSpec((_D, _D), lambda i: (0, 0)),
        pl.BlockSpec((1, _D), lambda i: (0, 0)),
        pl.BlockSpec((_D, _D), lambda i: (0, 0)),
        pl.BlockSpec((1, _D), lambda i: (0, 0)),
    ],
    out_specs=pl.BlockSpec((_NB, _D), lambda i: (i, 0)),
    out_shape=jax.ShapeDtypeStruct((_N, _D), jnp.float32),
)


def _pool_body(h_ref, bt_ref, wfc_ref, bfc_ref, o_ref):
    i = pl.program_id(0)
    bv = bt_ref[...].reshape(1, _NB)
    gid = lax.broadcasted_iota(jnp.int32, (_G, _NB), 0)
    oh = (gid == bv).astype(jnp.float32)
    gp = jnp.dot(oh, h_ref[...], preferred_element_type=jnp.float32)
    contrib = jnp.dot(gp, wfc_ref[...], preferred_element_type=jnp.float32)

    @pl.when(i == 0)
    def _():
        o_ref[...] = contrib + bfc_ref[...]

    @pl.when(i != 0)
    def _():
        o_ref[...] += contrib


_pool_call = pl.pallas_call(
    _pool_body,
    grid=(_N // _NB,),
    in_specs=[
        pl.BlockSpec((_NB, _D), lambda i: (i, 0)),
        pl.BlockSpec((1, 1, _NB), lambda i: (i, 0, 0)),
        pl.BlockSpec((_D, _OUT), lambda i: (0, 0)),
        pl.BlockSpec((1, _OUT), lambda i: (0, 0)),
    ],
    out_specs=pl.BlockSpec((_G, _OUT), lambda i: (0, 0)),
    out_shape=jax.ShapeDtypeStruct((_G, _OUT), jnp.float32),
)


def kernel(x, edge_index, edge_attr, batch, W_enc, b_enc, eps_all,
           W_edge_all, b_edge_all, W1_all, b1_all, W2_all, b2_all,
           W_fc, b_fc):
    src = edge_index[0].astype(jnp.int32)
    dst = edge_index[1].astype(jnp.int32)
    src_r = src.reshape(_NT, _CH // _IG, _IG, _B)
    dst_r = dst.reshape(_NT, _CH // _IG, _IG, _B)
    batch_r = batch.astype(jnp.int32).reshape(_N // _NB, 1, _NB)

    h = _enc_call(x, W_enc, b_enc.reshape(1, _D))
    for i in range(_L):
        e = _edge_call(edge_attr, W_edge_all[i], b_edge_all[i].reshape(1, _D))
        agg = _sc_agg(h, e, src_r, dst_r)
        h = _mlp_call(h, agg, (1.0 + eps_all[i]).reshape(1, 1),
                      W1_all[i], b1_all[i].reshape(1, _D),
                      W2_all[i], b2_all[i].reshape(1, _D))
    return _pool_call(h, batch_r, W_fc, b_fc.reshape(1, _OUT))


# speedup vs baseline: 1.0428x; 1.0428x over previous
import functools

import jax
import jax.numpy as jnp
from jax import lax
from jax.experimental import pallas as pl
from jax.experimental.pallas import tpu as pltpu
from jax.experimental.pallas import tpu_sc as plsc

_N = 10000
_E = 320000
_D = 128
_EDGE_D = 16
_G = 64
_L = 3
_OUT = 128

_NC = 2
_NS = 16
_NT = _NC * _NS
_B = 80
_CH = _E // (_NT * _B)
_IG = 25

_RPT = 624
_XB = 16
_LANES = 16


def _sc_agg_body(h_hbm, e_hbm, src_hbm, dst_hbm, out_hbm,
                 src_v, dst_v, rows_a, rows_b, msg_a, msg_b, agg_sh,
                 sga, sgb, sea, seb, ssa, ssb):
    c = lax.axis_index("c")
    s = lax.axis_index("s")
    tid = c * _NS + s

    zv = jnp.zeros((_LANES,), jnp.float32)

    def _zrow(r, carry):
        for j in range(_D // _LANES):
            rows_a[r, pl.ds(j * _LANES, _LANES)] = zv
        return carry

    lax.fori_loop(0, _B, _zrow, 0)
    start = s * _RPT
    for q in range(7):
        pltpu.sync_copy(rows_a, agg_sh.at[pl.ds(start + q * _B, _B)])

    @pl.when(s < _NS - 1)
    def _():
        pltpu.sync_copy(rows_a.at[pl.ds(0, 64)],
                        agg_sh.at[pl.ds(start + 7 * _B, 64)])

    @pl.when(s == _NS - 1)
    def _():
        pltpu.sync_copy(rows_a, agg_sh.at[pl.ds(start + 7 * _B, _B)])

    plsc.subcore_barrier()

    base = tid * (_CH * _B)

    def _eslice(g, k):
        return e_hbm.at[pl.ds(base + (g * _IG + k) * _B, _B)]

    def _gissue(g, k, rows_ref, sg):
        pltpu.async_copy(h_hbm.at[src_v.at[k]], rows_ref, sg)

    def _eissue(g, k, e_ref, se):
        pltpu.async_copy(_eslice(g, k), e_ref, se)

    def _drain(g, k, rows_ref, e_ref, sg, se):
        pltpu.make_async_copy(h_hbm.at[src_v.at[k]], rows_ref, sg).wait()
        pltpu.make_async_copy(_eslice(g, k), e_ref, se).wait()

        def _row(r, rcarry):
            for j in range(_D // _LANES):
                sl = pl.ds(j * _LANES, _LANES)
                e_ref[r, sl] = jnp.maximum(e_ref[r, sl] + rows_ref[r, sl],
                                           0.0)
            return rcarry

        lax.fori_loop(0, _B, _row, 0)
        pltpu.sync_copy(e_ref, agg_sh.at[dst_v.at[k]], add=True)

    def _iblock(g, carry):
        pltpu.sync_copy(src_hbm.at[tid, g], src_v)
        pltpu.sync_copy(dst_hbm.at[tid, g], dst_v)
        _gissue(g, 0, rows_a, sga)
        _eissue(g, 0, msg_a, sea)

        def _pair(m, kcarry):
            k0 = 2 * m
            _gissue(g, k0 + 1, rows_b, sgb)
            _eissue(g, k0 + 1, msg_b, seb)
            _drain(g, k0, rows_a, msg_a, sga, sea)
            _gissue(g, k0 + 2, rows_a, sga)
            _eissue(g, k0 + 2, msg_a, sea)
            _drain(g, k0 + 1, rows_b, msg_b, sgb, seb)
            return kcarry

        lax.fori_loop(0, (_IG - 1) // 2, _pair, 0)
        _drain(g, _IG - 1, rows_a, msg_a, sga, sea)
        return carry

    lax.fori_loop(0, _CH // _IG, _iblock, 0)
    plsc.subcore_barrier()

    for q in range(7):
        r0 = start + q * _B
        pltpu.sync_copy(agg_sh.at[pl.ds(r0, _B)], rows_a)
        pltpu.sync_copy(rows_a, out_hbm.at[c, pl.ds(r0, _B)])

    @pl.when(s < _NS - 1)
    def _():
        r0 = start + 7 * _B
        pltpu.sync_copy(agg_sh.at[pl.ds(r0, 64)], rows_a.at[pl.ds(0, 64)])
        pltpu.sync_copy(rows_a.at[pl.ds(0, 64)], out_hbm.at[c, pl.ds(r0, 64)])

    @pl.when(s == _NS - 1)
    def _():
        r0 = start + 7 * _B
        pltpu.sync_copy(agg_sh.at[pl.ds(r0, _B)], rows_a)
        pltpu.sync_copy(rows_a, out_hbm.at[c, pl.ds(r0, _B)])


_sc_agg = pl.kernel(
    _sc_agg_body,
    out_type=jax.ShapeDtypeStruct((_NC, _N, _D), jnp.float32),
    mesh=plsc.VectorSubcoreMesh(core_axis_name="c", subcore_axis_name="s"),
    scratch_types=[
        pltpu.VMEM((_IG, _B), jnp.int32),
        pltpu.VMEM((_IG, _B), jnp.int32),
        pltpu.VMEM((_B, _D), jnp.float32),
        pltpu.VMEM((_B, _D), jnp.float32),
        pltpu.VMEM((_B, _D), jnp.float32),
        pltpu.VMEM((_B, _D), jnp.float32),
        pltpu.VMEM_SHARED((_N, _D), jnp.float32),
        pltpu.SemaphoreType.DMA,
        pltpu.SemaphoreType.DMA,
        pltpu.SemaphoreType.DMA,
        pltpu.SemaphoreType.DMA,
        pltpu.SemaphoreType.DMA,
        pltpu.SemaphoreType.DMA,
    ],
)


_NB = 1000
_EB = 4000


def _enc_body(x_ref, w_ref, b_ref, o_ref):
    o_ref[...] = jnp.dot(x_ref[...], w_ref[...],
                         preferred_element_type=jnp.float32) + b_ref[...]


_enc_call = pl.pallas_call(
    _enc_body,
    grid=(_N // _NB,),
    in_specs=[
        pl.BlockSpec((_NB, _D), lambda i: (i, 0)),
        pl.BlockSpec((_D, _D), lambda i: (0, 0)),
        pl.BlockSpec((1, _D), lambda i: (0, 0)),
    ],
    out_specs=pl.BlockSpec((_NB, _D), lambda i: (i, 0)),
    out_shape=jax.ShapeDtypeStruct((_N, _D), jnp.float32),
)


def _edge_body(a_ref, w_ref, b_ref, o_ref):
    o_ref[...] = jnp.dot(a_ref[...], w_ref[...],
                         preferred_element_type=jnp.float32) + b_ref[...]


_edge_call = pl.pallas_call(
    _edge_body,
    grid=(_E // _EB,),
    in_specs=[
        pl.BlockSpec((_EB, _EDGE_D), lambda i: (i, 0)),
        pl.BlockSpec((_EDGE_D, _D), lambda i: (0, 0)),
        pl.BlockSpec((1, _D), lambda i: (0, 0)),
    ],
    out_specs=pl.BlockSpec((_EB, _D), lambda i: (i, 0)),
    out_shape=jax.ShapeDtypeStruct((_E, _D), jnp.float32),
)


def _mlp_body(h_ref, a_ref, s_ref, w1_ref, b1_ref, w2_ref, b2_ref, o_ref):
    scale = s_ref[0, 0]
    z = h_ref[...] * scale + a_ref[0] + a_ref[1]
    z = jnp.maximum(
        jnp.dot(z, w1_ref[...], preferred_element_type=jnp.float32)
        + b1_ref[...], 0.0)
    o_ref[...] = jnp.maximum(
        jnp.dot(z, w2_ref[...], preferred_element_type=jnp.float32)
        + b2_ref[...], 0.0)


_mlp_call = pl.pallas_call(
    _mlp_body,
    grid=(_N // _NB,),
    in_specs=[
        pl.BlockSpec((_NB, _D), lambda i: (i, 0)),
        pl.BlockSpec((_NC, _NB, _D), lambda i: (0, i, 0)),
        pl.BlockSpec((1, 1), lambda i: (0, 0)),
        pl.BlockSpec((_D, _D), lambda i: (0, 0)),
        pl.BlockSpec((1, _D), lambda i: (0, 0)),
        pl.BlockSpec((_D, _D), lambda i: (0, 0)),
        pl.BlockSpec((1, _D), lambda i: (0, 0)),
    ],
    out_specs=pl.BlockSpec((_NB, _D), lambda i: (i, 0)),
    out_shape=jax.ShapeDtypeStruct((_N, _D), jnp.float32),
)


def _pool_body(h_ref, bt_ref, wfc_ref, bfc_ref, o_ref):
    i = pl.program_id(0)
    bv = bt_ref[...].reshape(1, _NB)
    gid = lax.broadcasted_iota(jnp.int32, (_G, _NB), 0)
    oh = (gid == bv).astype(jnp.float32)
    gp = jnp.dot(oh, h_ref[...], preferred_element_type=jnp.float32)
    contrib = jnp.dot(gp, wfc_ref[...], preferred_element_type=jnp.float32)

    @pl.when(i == 0)
    def _():
        o_ref[...] = contrib + bfc_ref[...]

    @pl.when(i != 0)
    def _():
        o_ref[...] += contrib


_pool_call = pl.pallas_call(
    _pool_body,
    grid=(_N // _NB,),
    in_specs=[
        pl.BlockSpec((_NB, _D), lambda i: (i, 0)),
        pl.BlockSpec((1, 1, _NB), lambda i: (i, 0, 0)),
        pl.BlockSpec((_D, _OUT), lambda i: (0, 0)),
        pl.BlockSpec((1, _OUT), lambda i: (0, 0)),
    ],
    out_specs=pl.BlockSpec((_G, _OUT), lambda i: (0, 0)),
    out_shape=jax.ShapeDtypeStruct((_G, _OUT), jnp.float32),
)


def kernel(x, edge_index, edge_attr, batch, W_enc, b_enc, eps_all,
           W_edge_all, b_edge_all, W1_all, b1_all, W2_all, b2_all,
           W_fc, b_fc):
    src = edge_index[0].astype(jnp.int32)
    dst = edge_index[1].astype(jnp.int32)
    src_r = src.reshape(_NT, _CH // _IG, _IG, _B)
    dst_r = dst.reshape(_NT, _CH // _IG, _IG, _B)
    batch_r = batch.astype(jnp.int32).reshape(_N // _NB, 1, _NB)

    h = _enc_call(x, W_enc, b_enc.reshape(1, _D))
    for i in range(_L):
        e = _edge_call(edge_attr, W_edge_all[i], b_edge_all[i].reshape(1, _D))
        agg = _sc_agg(h, e, src_r, dst_r)
        h = _mlp_call(h, agg, (1.0 + eps_all[i]).reshape(1, 1),
                      W1_all[i], b1_all[i].reshape(1, _D),
                      W2_all[i], b2_all[i].reshape(1, _D))
    return _pool_call(h, batch_r, W_fc, b_fc.reshape(1, _OUT))


# speedup vs baseline: 1.0436x; 1.0008x over previous
import functools

import jax
import jax.numpy as jnp
from jax import lax
from jax.experimental import pallas as pl
from jax.experimental.pallas import tpu as pltpu
from jax.experimental.pallas import tpu_sc as plsc

_N = 10000
_E = 320000
_D = 128
_EDGE_D = 16
_G = 64
_L = 3
_OUT = 128

_NC = 2
_NS = 16
_NT = _NC * _NS
_B = 80
_CH = _E // (_NT * _B)
_IG = 25

_RPT = 624
_XB = 16
_LANES = 16


def _sc_agg_body(h_hbm, e_hbm, src_hbm, dst_hbm, out_hbm,
                 src_v, dst_v, rows_a, rows_b, msg_a, msg_b, agg_sh,
                 sga, sgb, sea, seb):
    c = lax.axis_index("c")
    s = lax.axis_index("s")
    tid = c * _NS + s

    zv = jnp.zeros((_LANES,), jnp.float32)

    def _zrow(r, carry):
        for j in range(_D // _LANES):
            rows_a[r, pl.ds(j * _LANES, _LANES)] = zv
        return carry

    lax.fori_loop(0, _B, _zrow, 0)
    start = s * _RPT
    for q in range(7):
        pltpu.sync_copy(rows_a, agg_sh.at[pl.ds(start + q * _B, _B)])

    @pl.when(s < _NS - 1)
    def _():
        pltpu.sync_copy(rows_a.at[pl.ds(0, 64)],
                        agg_sh.at[pl.ds(start + 7 * _B, 64)])

    @pl.when(s == _NS - 1)
    def _():
        pltpu.sync_copy(rows_a, agg_sh.at[pl.ds(start + 7 * _B, _B)])

    plsc.subcore_barrier()

    base = tid * (_CH * _B)

    def _eslice(g, k):
        return e_hbm.at[pl.ds(base + (g * _IG + k) * _B, _B)]

    def _gissue(g, k, rows_ref, sg):
        pltpu.async_copy(h_hbm.at[src_v.at[k]], rows_ref, sg)

    def _eissue(g, k, e_ref, se):
        pltpu.async_copy(_eslice(g, k), e_ref, se)

    def _drain(g, k, rows_ref, e_ref, sg, se):
        pltpu.make_async_copy(h_hbm.at[src_v.at[k]], rows_ref, sg).wait()
        pltpu.make_async_copy(_eslice(g, k), e_ref, se).wait()

        def _row(r, rcarry):
            for j in range(_D // _LANES):
                sl = pl.ds(j * _LANES, _LANES)
                e_ref[r, sl] = jnp.maximum(e_ref[r, sl] + rows_ref[r, sl],
                                           0.0)
            return rcarry

        lax.fori_loop(0, _B, _row, 0)
        pltpu.sync_copy(e_ref, agg_sh.at[dst_v.at[k]], add=True)

    def _iblock(g, carry):
        pltpu.sync_copy(src_hbm.at[tid, g], src_v)
        pltpu.sync_copy(dst_hbm.at[tid, g], dst_v)
        _gissue(g, 0, rows_a, sga)
        _eissue(g, 0, msg_a, sea)

        def _pair(m, kcarry):
            k0 = 2 * m
            _gissue(g, k0 + 1, rows_b, sgb)
            _eissue(g, k0 + 1, msg_b, seb)
            _drain(g, k0, rows_a, msg_a, sga, sea)
            _gissue(g, k0 + 2, rows_a, sga)
            _eissue(g, k0 + 2, msg_a, sea)
            _drain(g, k0 + 1, rows_b, msg_b, sgb, seb)
            return kcarry

        lax.fori_loop(0, (_IG - 1) // 2, _pair, 0)
        _drain(g, _IG - 1, rows_a, msg_a, sga, sea)
        return carry

    lax.fori_loop(0, _CH // _IG, _iblock, 0)
    plsc.subcore_barrier()

    for q in range(7):
        r0 = start + q * _B
        pltpu.sync_copy(agg_sh.at[pl.ds(r0, _B)], rows_a)
        pltpu.sync_copy(rows_a, out_hbm.at[c, pl.ds(r0, _B)])

    @pl.when(s < _NS - 1)
    def _():
        r0 = start + 7 * _B
        pltpu.sync_copy(agg_sh.at[pl.ds(r0, 64)], rows_a.at[pl.ds(0, 64)])
        pltpu.sync_copy(rows_a.at[pl.ds(0, 64)], out_hbm.at[c, pl.ds(r0, 64)])

    @pl.when(s == _NS - 1)
    def _():
        r0 = start + 7 * _B
        pltpu.sync_copy(agg_sh.at[pl.ds(r0, _B)], rows_a)
        pltpu.sync_copy(rows_a, out_hbm.at[c, pl.ds(r0, _B)])


_sc_agg = pl.kernel(
    _sc_agg_body,
    out_type=jax.ShapeDtypeStruct((_NC, _N, _D), jnp.float32),
    mesh=plsc.VectorSubcoreMesh(core_axis_name="c", subcore_axis_name="s"),
    scratch_types=[
        pltpu.VMEM((_IG, _B), jnp.int32),
        pltpu.VMEM((_IG, _B), jnp.int32),
        pltpu.VMEM((_B, _D), jnp.float32),
        pltpu.VMEM((_B, _D), jnp.float32),
        pltpu.VMEM((_B, _D), jnp.float32),
        pltpu.VMEM((_B, _D), jnp.float32),
        pltpu.VMEM_SHARED((_N, _D), jnp.float32),
        pltpu.SemaphoreType.DMA,
        pltpu.SemaphoreType.DMA,
        pltpu.SemaphoreType.DMA,
        pltpu.SemaphoreType.DMA,
    ],
)


_NB = 1000
_EB = 4000


def _enc_body(x_ref, w_ref, b_ref, o_ref):
    o_ref[...] = jnp.dot(x_ref[...], w_ref[...],
                         preferred_element_type=jnp.float32) + b_ref[...]


_enc_call = pl.pallas_call(
    _enc_body,
    grid=(_N // _NB,),
    in_specs=[
        pl.BlockSpec((_NB, _D), lambda i: (i, 0)),
        pl.BlockSpec((_D, _D), lambda i: (0, 0)),
        pl.BlockSpec((1, _D), lambda i: (0, 0)),
    ],
    out_specs=pl.BlockSpec((_NB, _D), lambda i: (i, 0)),
    out_shape=jax.ShapeDtypeStruct((_N, _D), jnp.float32),
)


def _edge_body(a_ref, w_ref, b_ref, o_ref):
    o_ref[...] = jnp.dot(a_ref[...], w_ref[...],
                         preferred_element_type=jnp.float32) + b_ref[...]


_edge_call = pl.pallas_call(
    _edge_body,
    grid=(_E // _EB,),
    in_specs=[
        pl.BlockSpec((_EB, _EDGE_D), lambda i: (i, 0)),
        pl.BlockSpec((_EDGE_D, _D), lambda i: (0, 0)),
        pl.BlockSpec((1, _D), lambda i: (0, 0)),
    ],
    out_specs=pl.BlockSpec((_EB, _D), lambda i: (i, 0)),
    out_shape=jax.ShapeDtypeStruct((_E, _D), jnp.float32),
)


def _mlp_body(h_ref, a_ref, s_ref, w1_ref, b1_ref, w2_ref, b2_ref, o_ref):
    scale = s_ref[0, 0]
    z = h_ref[...] * scale + a_ref[0] + a_ref[1]
    z = jnp.maximum(
        jnp.dot(z, w1_ref[...], preferred_element_type=jnp.float32)
        + b1_ref[...], 0.0)
    o_ref[...] = jnp.maximum(
        jnp.dot(z, w2_ref[...], preferred_element_type=jnp.float32)
        + b2_ref[...], 0.0)


_mlp_call = pl.pallas_call(
    _mlp_body,
    grid=(_N // _NB,),
    in_specs=[
        pl.BlockSpec((_NB, _D), lambda i: (i, 0)),
        pl.BlockSpec((_NC, _NB, _D), lambda i: (0, i, 0)),
        pl.BlockSpec((1, 1), lambda i: (0, 0)),
        pl.BlockSpec((_D, _D), lambda i: (0, 0)),
        pl.BlockSpec((1, _D), lambda i: (0, 0)),
        pl.BlockSpec((_D, _D), lambda i: (0, 0)),
        pl.BlockSpec((1, _D), lambda i: (0, 0)),
    ],
    out_specs=pl.BlockSpec((_NB, _D), lambda i: (i, 0)),
    out_shape=jax.ShapeDtypeStruct((_N, _D), jnp.float32),
)


def _pool_body(h_ref, bt_ref, wfc_ref, bfc_ref, o_ref):
    i = pl.program_id(0)
    bv = bt_ref[...].reshape(1, _NB)
    gid = lax.broadcasted_iota(jnp.int32, (_G, _NB), 0)
    oh = (gid == bv).astype(jnp.float32)
    gp = jnp.dot(oh, h_ref[...], preferred_element_type=jnp.float32)
    contrib = jnp.dot(gp, wfc_ref[...], preferred_element_type=jnp.float32)

    @pl.when(i == 0)
    def _():
        o_ref[...] = contrib + bfc_ref[...]

    @pl.when(i != 0)
    def _():
        o_ref[...] += contrib


_pool_call = pl.pallas_call(
    _pool_body,
    grid=(_N // _NB,),
    in_specs=[
        pl.BlockSpec((_NB, _D), lambda i: (i, 0)),
        pl.BlockSpec((1, 1, _NB), lambda i: (i, 0, 0)),
        pl.BlockSpec((_D, _OUT), lambda i: (0, 0)),
        pl.BlockSpec((1, _OUT), lambda i: (0, 0)),
    ],
    out_specs=pl.BlockSpec((_G, _OUT), lambda i: (0, 0)),
    out_shape=jax.ShapeDtypeStruct((_G, _OUT), jnp.float32),
)


def kernel(x, edge_index, edge_attr, batch, W_enc, b_enc, eps_all,
           W_edge_all, b_edge_all, W1_all, b1_all, W2_all, b2_all,
           W_fc, b_fc):
    src = edge_index[0].astype(jnp.int32)
    dst = edge_index[1].astype(jnp.int32)
    src_r = src.reshape(_NT, _CH // _IG, _IG, _B)
    dst_r = dst.reshape(_NT, _CH // _IG, _IG, _B)
    batch_r = batch.astype(jnp.int32).reshape(_N // _NB, 1, _NB)

    h = _enc_call(x, W_enc, b_enc.reshape(1, _D))
    es = [_edge_call(edge_attr, W_edge_all[i], b_edge_all[i].reshape(1, _D))
          for i in range(_L)]
    for i in range(_L):
        agg = _sc_agg(h, es[i], src_r, dst_r)
        h = _mlp_call(h, agg, (1.0 + eps_all[i]).reshape(1, 1),
                      W1_all[i], b1_all[i].reshape(1, _D),
                      W2_all[i], b2_all[i].reshape(1, _D))
    return _pool_call(h, batch_r, W_fc, b_fc.reshape(1, _OUT))


# speedup vs baseline: 1.0489x; 1.0051x over previous
import functools

import jax
import jax.numpy as jnp
from jax import lax
from jax.experimental import pallas as pl
from jax.experimental.pallas import tpu as pltpu
from jax.experimental.pallas import tpu_sc as plsc

_N = 10000
_E = 320000
_D = 128
_EDGE_D = 16
_G = 64
_L = 3
_OUT = 128

_NC = 2
_NS = 16
_NT = _NC * _NS
_B = 80
_CH = _E // (_NT * _B)
_IG = 25

_RPT = 624
_XB = 16
_LANES = 16


def _sc_agg_body(h_hbm, e_hbm, src_hbm, dst_hbm, out_hbm,
                 src_v, dst_v, rows_a, rows_b, msg_a, msg_b, agg_sh,
                 sga, sgb, sea, seb):
    c = lax.axis_index("c")
    s = lax.axis_index("s")
    tid = c * _NS + s

    zv = jnp.zeros((_LANES,), jnp.float32)

    def _zrow(r, carry):
        for j in range(_D // _LANES):
            rows_a[r, pl.ds(j * _LANES, _LANES)] = zv
        return carry

    lax.fori_loop(0, _B, _zrow, 0)
    start = s * _RPT
    for q in range(7):
        pltpu.async_copy(rows_a, agg_sh.at[pl.ds(start + q * _B, _B)], sga)

    @pl.when(s < _NS - 1)
    def _():
        pltpu.async_copy(rows_a.at[pl.ds(0, 64)],
                         agg_sh.at[pl.ds(start + 7 * _B, 64)], sga)

    @pl.when(s == _NS - 1)
    def _():
        pltpu.async_copy(rows_a, agg_sh.at[pl.ds(start + 7 * _B, _B)], sga)

    for q in range(7):
        pltpu.make_async_copy(
            rows_a, agg_sh.at[pl.ds(start + q * _B, _B)], sga).wait()

    @pl.when(s < _NS - 1)
    def _():
        pltpu.make_async_copy(rows_a.at[pl.ds(0, 64)],
                              agg_sh.at[pl.ds(start + 7 * _B, 64)], sga).wait()

    @pl.when(s == _NS - 1)
    def _():
        pltpu.make_async_copy(rows_a,
                              agg_sh.at[pl.ds(start + 7 * _B, _B)], sga).wait()

    plsc.subcore_barrier()

    base = tid * (_CH * _B)

    def _eslice(g, k):
        return e_hbm.at[pl.ds(base + (g * _IG + k) * _B, _B)]

    def _gissue(g, k, rows_ref, sg):
        pltpu.async_copy(h_hbm.at[src_v.at[k]], rows_ref, sg)

    def _eissue(g, k, e_ref, se):
        pltpu.async_copy(_eslice(g, k), e_ref, se)

    def _drain(g, k, rows_ref, e_ref, sg, se):
        pltpu.make_async_copy(h_hbm.at[src_v.at[k]], rows_ref, sg).wait()
        pltpu.make_async_copy(_eslice(g, k), e_ref, se).wait()

        def _row(r, rcarry):
            for j in range(_D // _LANES):
                sl = pl.ds(j * _LANES, _LANES)
                e_ref[r, sl] = jnp.maximum(e_ref[r, sl] + rows_ref[r, sl],
                                           0.0)
            return rcarry

        lax.fori_loop(0, _B, _row, 0)
        pltpu.sync_copy(e_ref, agg_sh.at[dst_v.at[k]], add=True)

    def _iblock(g, carry):
        pltpu.sync_copy(src_hbm.at[tid, g], src_v)
        pltpu.sync_copy(dst_hbm.at[tid, g], dst_v)
        _gissue(g, 0, rows_a, sga)
        _eissue(g, 0, msg_a, sea)

        def _pair(m, kcarry):
            k0 = 2 * m
            _gissue(g, k0 + 1, rows_b, sgb)
            _eissue(g, k0 + 1, msg_b, seb)
            _drain(g, k0, rows_a, msg_a, sga, sea)
            _gissue(g, k0 + 2, rows_a, sga)
            _eissue(g, k0 + 2, msg_a, sea)
            _drain(g, k0 + 1, rows_b, msg_b, sgb, seb)
            return kcarry

        lax.fori_loop(0, (_IG - 1) // 2, _pair, 0)
        _drain(g, _IG - 1, rows_a, msg_a, sga, sea)
        return carry

    lax.fori_loop(0, _CH // _IG, _iblock, 0)
    plsc.subcore_barrier()

    bufs = (rows_a, rows_b)
    sems = (sga, sgb)
    for q in range(7):
        buf = bufs[q % 2]
        sm = sems[q % 2]
        r0 = start + q * _B
        if q >= 2:
            pltpu.make_async_copy(
                buf, out_hbm.at[c, pl.ds(start + (q - 2) * _B, _B)],
                sm).wait()
        pltpu.sync_copy(agg_sh.at[pl.ds(r0, _B)], buf)
        pltpu.async_copy(buf, out_hbm.at[c, pl.ds(r0, _B)], sm)

    pltpu.make_async_copy(rows_b, out_hbm.at[c, pl.ds(start + 5 * _B, _B)],
                          sgb).wait()
    r7 = start + 7 * _B

    @pl.when(s < _NS - 1)
    def _():
        pltpu.sync_copy(agg_sh.at[pl.ds(r7, 64)], rows_b.at[pl.ds(0, 64)])
        pltpu.async_copy(rows_b.at[pl.ds(0, 64)], out_hbm.at[c, pl.ds(r7, 64)],
                         sgb)

    @pl.when(s == _NS - 1)
    def _():
        pltpu.sync_copy(agg_sh.at[pl.ds(r7, _B)], rows_b)
        pltpu.async_copy(rows_b, out_hbm.at[c, pl.ds(r7, _B)], sgb)

    pltpu.make_async_copy(rows_a, out_hbm.at[c, pl.ds(start + 6 * _B, _B)],
                          sga).wait()

    @pl.when(s < _NS - 1)
    def _():
        pltpu.make_async_copy(rows_b.at[pl.ds(0, 64)],
                              out_hbm.at[c, pl.ds(r7, 64)], sgb).wait()

    @pl.when(s == _NS - 1)
    def _():
        pltpu.make_async_copy(rows_b, out_hbm.at[c, pl.ds(r7, _B)],
                              sgb).wait()


_sc_agg = pl.kernel(
    _sc_agg_body,
    out_type=jax.ShapeDtypeStruct((_NC, _N, _D), jnp.float32),
    mesh=plsc.VectorSubcoreMesh(core_axis_name="c", subcore_axis_name="s"),
    scratch_types=[
        pltpu.VMEM((_IG, _B), jnp.int32),
        pltpu.VMEM((_IG, _B), jnp.int32),
        pltpu.VMEM((_B, _D), jnp.float32),
        pltpu.VMEM((_B, _D), jnp.float32),
        pltpu.VMEM((_B, _D), jnp.float32),
        pltpu.VMEM((_B, _D), jnp.float32),
        pltpu.VMEM_SHARED((_N, _D), jnp.float32),
        pltpu.SemaphoreType.DMA,
        pltpu.SemaphoreType.DMA,
        pltpu.SemaphoreType.DMA,
        pltpu.SemaphoreType.DMA,
    ],
)


_NB = 1000
_EB = 4000


def _enc_body(x_ref, w_ref, b_ref, o_ref):
    o_ref[...] = jnp.dot(x_ref[...], w_ref[...],
                         preferred_element_type=jnp.float32) + b_ref[...]


_enc_call = pl.pallas_call(
    _enc_body,
    grid=(_N // _NB,),
    in_specs=[
        pl.BlockSpec((_NB, _D), lambda i: (i, 0)),
        pl.BlockSpec((_D, _D), lambda i: (0, 0)),
        pl.BlockSpec((1, _D), lambda i: (0, 0)),
    ],
    out_specs=pl.BlockSpec((_NB, _D), lambda i: (i, 0)),
    out_shape=jax.ShapeDtypeStruct((_N, _D), jnp.float32),
)


def _edge_body(a_ref, w_ref, b_ref, o_ref):
    o_ref[...] = jnp.dot(a_ref[...], w_ref[...],
                         preferred_element_type=jnp.float32) + b_ref[...]


_edge_call = pl.pallas_call(
    _edge_body,
    grid=(_E // _EB,),
    in_specs=[
        pl.BlockSpec((_EB, _EDGE_D), lambda i: (i, 0)),
        pl.BlockSpec((_EDGE_D, _D), lambda i: (0, 0)),
        pl.BlockSpec((1, _D), lambda i: (0, 0)),
    ],
    out_specs=pl.BlockSpec((_EB, _D), lambda i: (i, 0)),
    out_shape=jax.ShapeDtypeStruct((_E, _D), jnp.float32),
)


def _mlp_body(h_ref, a_ref, s_ref, w1_ref, b1_ref, w2_ref, b2_ref, o_ref):
    scale = s_ref[0, 0]
    z = h_ref[...] * scale + a_ref[0] + a_ref[1]
    z = jnp.maximum(
        jnp.dot(z, w1_ref[...], preferred_element_type=jnp.float32)
        + b1_ref[...], 0.0)
    o_ref[...] = jnp.maximum(
        jnp.dot(z, w2_ref[...], preferred_element_type=jnp.float32)
        + b2_ref[...], 0.0)


_mlp_call = pl.pallas_call(
    _mlp_body,
    grid=(_N // _NB,),
    in_specs=[
        pl.BlockSpec((_NB, _D), lambda i: (i, 0)),
        pl.BlockSpec((_NC, _NB, _D), lambda i: (0, i, 0)),
        pl.BlockSpec((1, 1), lambda i: (0, 0)),
        pl.BlockSpec((_D, _D), lambda i: (0, 0)),
        pl.BlockSpec((1, _D), lambda i: (0, 0)),
        pl.BlockSpec((_D, _D), lambda i: (0, 0)),
        pl.BlockSpec((1, _D), lambda i: (0, 0)),
    ],
    out_specs=pl.BlockSpec((_NB, _D), lambda i: (i, 0)),
    out_shape=jax.ShapeDtypeStruct((_N, _D), jnp.float32),
)


def _pool_body(h_ref, bt_ref, wfc_ref, bfc_ref, o_ref):
    i = pl.program_id(0)
    bv = bt_ref[...].reshape(1, _NB)
    gid = lax.broadcasted_iota(jnp.int32, (_G, _NB), 0)
    oh = (gid == bv).astype(jnp.float32)
    gp = jnp.dot(oh, h_ref[...], preferred_element_type=jnp.float32)
    contrib = jnp.dot(gp, wfc_ref[...], preferred_element_type=jnp.float32)

    @pl.when(i == 0)
    def _():
        o_ref[...] = contrib + bfc_ref[...]

    @pl.when(i != 0)
    def _():
        o_ref[...] += contrib


_pool_call = pl.pallas_call(
    _pool_body,
    grid=(_N // _NB,),
    in_specs=[
        pl.BlockSpec((_NB, _D), lambda i: (i, 0)),
        pl.BlockSpec((1, 1, _NB), lambda i: (i, 0, 0)),
        pl.BlockSpec((_D, _OUT), lambda i: (0, 0)),
        pl.BlockSpec((1, _OUT), lambda i: (0, 0)),
    ],
    out_specs=pl.BlockSpec((_G, _OUT), lambda i: (0, 0)),
    out_shape=jax.ShapeDtypeStruct((_G, _OUT), jnp.float32),
)


def kernel(x, edge_index, edge_attr, batch, W_enc, b_enc, eps_all,
           W_edge_all, b_edge_all, W1_all, b1_all, W2_all, b2_all,
           W_fc, b_fc):
    src = edge_index[0].astype(jnp.int32)
    dst = edge_index[1].astype(jnp.int32)
    src_r = src.reshape(_NT, _CH // _IG, _IG, _B)
    dst_r = dst.reshape(_NT, _CH // _IG, _IG, _B)
    batch_r = batch.astype(jnp.int32).reshape(_N // _NB, 1, _NB)

    h = _enc_call(x, W_enc, b_enc.reshape(1, _D))
    es = [_edge_call(edge_attr, W_edge_all[i], b_edge_all[i].reshape(1, _D))
          for i in range(_L)]
    for i in range(_L):
        agg = _sc_agg(h, es[i], src_r, dst_r)
        h = _mlp_call(h, agg, (1.0 + eps_all[i]).reshape(1, 1),
                      W1_all[i], b1_all[i].reshape(1, _D),
                      W2_all[i], b2_all[i].reshape(1, _D))
    return _pool_call(h, batch_r, W_fc, b_fc.reshape(1, _OUT))


# speedup vs baseline: 1.0517x; 1.0027x over previous
import functools

import jax
import jax.numpy as jnp
from jax import lax
from jax.experimental import pallas as pl
from jax.experimental.pallas import tpu as pltpu
from jax.experimental.pallas import tpu_sc as plsc

_N = 10000
_E = 320000
_D = 128
_EDGE_D = 16
_G = 64
_L = 3
_OUT = 128

_NC = 2
_NS = 16
_NT = _NC * _NS
_B = 80
_CH = _E // (_NT * _B)
_IG = 25

_RPT = 624
_XB = 16
_LANES = 16


def _sc_agg_body(h_hbm, e_hbm, src_hbm, dst_hbm, out_hbm,
                 src_v, dst_v, rows_a, rows_b, msg_a, msg_b, agg_sh,
                 sga, sgb, sea, seb):
    c = lax.axis_index("c")
    s = lax.axis_index("s")
    tid = c * _NS + s

    zv = jnp.zeros((_LANES,), jnp.float32)

    def _zrow(r, carry):
        for j in range(_D // _LANES):
            rows_a[r, pl.ds(j * _LANES, _LANES)] = zv
        return carry

    lax.fori_loop(0, _B, _zrow, 0)
    start = s * _RPT
    for q in range(7):
        pltpu.async_copy(rows_a, agg_sh.at[pl.ds(start + q * _B, _B)], sga)

    @pl.when(s < _NS - 1)
    def _():
        pltpu.async_copy(rows_a.at[pl.ds(0, 64)],
                         agg_sh.at[pl.ds(start + 7 * _B, 64)], sga)

    @pl.when(s == _NS - 1)
    def _():
        pltpu.async_copy(rows_a, agg_sh.at[pl.ds(start + 7 * _B, _B)], sga)

    for q in range(7):
        pltpu.make_async_copy(
            rows_a, agg_sh.at[pl.ds(start + q * _B, _B)], sga).wait()

    @pl.when(s < _NS - 1)
    def _():
        pltpu.make_async_copy(rows_a.at[pl.ds(0, 64)],
                              agg_sh.at[pl.ds(start + 7 * _B, 64)], sga).wait()

    @pl.when(s == _NS - 1)
    def _():
        pltpu.make_async_copy(rows_a,
                              agg_sh.at[pl.ds(start + 7 * _B, _B)], sga).wait()

    plsc.subcore_barrier()

    base = tid * (_CH * _B)

    def _eslice(g, k):
        return e_hbm.at[pl.ds(base + (g * _IG + k) * _B, _B)]

    def _gissue(g, k, rows_ref, sg):
        pltpu.async_copy(h_hbm.at[src_v.at[k]], rows_ref, sg)

    def _eissue(g, k, e_ref, se):
        pltpu.async_copy(_eslice(g, k), e_ref, se)

    def _drain(g, k, rows_ref, e_ref, sg, se):
        pltpu.make_async_copy(h_hbm.at[src_v.at[k]], rows_ref, sg).wait()
        pltpu.make_async_copy(_eslice(g, k), e_ref, se).wait()

        def _row(r2, rcarry):
            for rr in range(2):
                r = 2 * r2 + rr
                for j in range(_D // _LANES):
                    sl = pl.ds(j * _LANES, _LANES)
                    e_ref[r, sl] = jnp.maximum(
                        e_ref[r, sl] + rows_ref[r, sl], 0.0)
            return rcarry

        lax.fori_loop(0, _B // 2, _row, 0)
        pltpu.sync_copy(e_ref, agg_sh.at[dst_v.at[k]], add=True)

    def _iblock(g, carry):
        pltpu.sync_copy(src_hbm.at[tid, g], src_v)
        pltpu.sync_copy(dst_hbm.at[tid, g], dst_v)
        _gissue(g, 0, rows_a, sga)
        _eissue(g, 0, msg_a, sea)

        def _pair(m, kcarry):
            k0 = 2 * m
            _gissue(g, k0 + 1, rows_b, sgb)
            _eissue(g, k0 + 1, msg_b, seb)
            _drain(g, k0, rows_a, msg_a, sga, sea)
            _gissue(g, k0 + 2, rows_a, sga)
            _eissue(g, k0 + 2, msg_a, sea)
            _drain(g, k0 + 1, rows_b, msg_b, sgb, seb)
            return kcarry

        lax.fori_loop(0, (_IG - 1) // 2, _pair, 0)
        _drain(g, _IG - 1, rows_a, msg_a, sga, sea)
        return carry

    lax.fori_loop(0, _CH // _IG, _iblock, 0)
    plsc.subcore_barrier()

    bufs = (rows_a, rows_b)
    sems = (sga, sgb)
    for q in range(7):
        buf = bufs[q % 2]
        sm = sems[q % 2]
        r0 = start + q * _B
        if q >= 2:
            pltpu.make_async_copy(
                buf, out_hbm.at[c, pl.ds(start + (q - 2) * _B, _B)],
                sm).wait()
        pltpu.sync_copy(agg_sh.at[pl.ds(r0, _B)], buf)
        pltpu.async_copy(buf, out_hbm.at[c, pl.ds(r0, _B)], sm)

    pltpu.make_async_copy(rows_b, out_hbm.at[c, pl.ds(start + 5 * _B, _B)],
                          sgb).wait()
    r7 = start + 7 * _B

    @pl.when(s < _NS - 1)
    def _():
        pltpu.sync_copy(agg_sh.at[pl.ds(r7, 64)], rows_b.at[pl.ds(0, 64)])
        pltpu.async_copy(rows_b.at[pl.ds(0, 64)], out_hbm.at[c, pl.ds(r7, 64)],
                         sgb)

    @pl.when(s == _NS - 1)
    def _():
        pltpu.sync_copy(agg_sh.at[pl.ds(r7, _B)], rows_b)
        pltpu.async_copy(rows_b, out_hbm.at[c, pl.ds(r7, _B)], sgb)

    pltpu.make_async_copy(rows_a, out_hbm.at[c, pl.ds(start + 6 * _B, _B)],
                          sga).wait()

    @pl.when(s < _NS - 1)
    def _():
        pltpu.make_async_copy(rows_b.at[pl.ds(0, 64)],
                              out_hbm.at[c, pl.ds(r7, 64)], sgb).wait()

    @pl.when(s == _NS - 1)
    def _():
        pltpu.make_async_copy(rows_b, out_hbm.at[c, pl.ds(r7, _B)],
                              sgb).wait()


_sc_agg = pl.kernel(
    _sc_agg_body,
    out_type=jax.ShapeDtypeStruct((_NC, _N, _D), jnp.float32),
    mesh=plsc.VectorSubcoreMesh(core_axis_name="c", subcore_axis_name="s"),
    scratch_types=[
        pltpu.VMEM((_IG, _B), jnp.int32),
        pltpu.VMEM((_IG, _B), jnp.int32),
        pltpu.VMEM((_B, _D), jnp.float32),
        pltpu.VMEM((_B, _D), jnp.float32),
        pltpu.VMEM((_B, _D), jnp.float32),
        pltpu.VMEM((_B, _D), jnp.float32),
        pltpu.VMEM_SHARED((_N, _D), jnp.float32),
        pltpu.SemaphoreType.DMA,
        pltpu.SemaphoreType.DMA,
        pltpu.SemaphoreType.DMA,
        pltpu.SemaphoreType.DMA,
    ],
)


_NB = 1000
_EB = 4000


def _enc_body(x_ref, w_ref, b_ref, o_ref):
    o_ref[...] = jnp.dot(x_ref[...], w_ref[...],
                         preferred_element_type=jnp.float32) + b_ref[...]


_enc_call = pl.pallas_call(
    _enc_body,
    grid=(_N // _NB,),
    in_specs=[
        pl.BlockSpec((_NB, _D), lambda i: (i, 0)),
        pl.BlockSpec((_D, _D), lambda i: (0, 0)),
        pl.BlockSpec((1, _D), lambda i: (0, 0)),
    ],
    out_specs=pl.BlockSpec((_NB, _D), lambda i: (i, 0)),
    out_shape=jax.ShapeDtypeStruct((_N, _D), jnp.float32),
)


def _edge_body(a_ref, w_ref, b_ref, o_ref):
    o_ref[...] = jnp.dot(a_ref[...], w_ref[...],
                         preferred_element_type=jnp.float32) + b_ref[...]


_edge_call = pl.pallas_call(
    _edge_body,
    grid=(_E // _EB,),
    in_specs=[
        pl.BlockSpec((_EB, _EDGE_D), lambda i: (i, 0)),
        pl.BlockSpec((_EDGE_D, _D), lambda i: (0, 0)),
        pl.BlockSpec((1, _D), lambda i: (0, 0)),
    ],
    out_specs=pl.BlockSpec((_EB, _D), lambda i: (i, 0)),
    out_shape=jax.ShapeDtypeStruct((_E, _D), jnp.float32),
)


def _mlp_body(h_ref, a_ref, s_ref, w1_ref, b1_ref, w2_ref, b2_ref, o_ref):
    scale = s_ref[0, 0]
    z = h_ref[...] * scale + a_ref[0] + a_ref[1]
    z = jnp.maximum(
        jnp.dot(z, w1_ref[...], preferred_element_type=jnp.float32)
        + b1_ref[...], 0.0)
    o_ref[...] = jnp.maximum(
        jnp.dot(z, w2_ref[...], preferred_element_type=jnp.float32)
        + b2_ref[...], 0.0)


_mlp_call = pl.pallas_call(
    _mlp_body,
    grid=(_N // _NB,),
    in_specs=[
        pl.BlockSpec((_NB, _D), lambda i: (i, 0)),
        pl.BlockSpec((_NC, _NB, _D), lambda i: (0, i, 0)),
        pl.BlockSpec((1, 1), lambda i: (0, 0)),
        pl.BlockSpec((_D, _D), lambda i: (0, 0)),
        pl.BlockSpec((1, _D), lambda i: (0, 0)),
        pl.BlockSpec((_D, _D), lambda i: (0, 0)),
        pl.BlockSpec((1, _D), lambda i: (0, 0)),
    ],
    out_specs=pl.BlockSpec((_NB, _D), lambda i: (i, 0)),
    out_shape=jax.ShapeDtypeStruct((_N, _D), jnp.float32),
)


def _pool_body(h_ref, bt_ref, wfc_ref, bfc_ref, o_ref):
    i = pl.program_id(0)
    bv = bt_ref[...].reshape(1, _NB)
    gid = lax.broadcasted_iota(jnp.int32, (_G, _NB), 0)
    oh = (gid == bv).astype(jnp.float32)
    gp = jnp.dot(oh, h_ref[...], preferred_element_type=jnp.float32)
    contrib = jnp.dot(gp, wfc_ref[...], preferred_element_type=jnp.float32)

    @pl.when(i == 0)
    def _():
        o_ref[...] = contrib + bfc_ref[...]

    @pl.when(i != 0)
    def _():
        o_ref[...] += contrib


_pool_call = pl.pallas_call(
    _pool_body,
    grid=(_N // _NB,),
    in_specs=[
        pl.BlockSpec((_NB, _D), lambda i: (i, 0)),
        pl.BlockSpec((1, 1, _NB), lambda i: (i, 0, 0)),
        pl.BlockSpec((_D, _OUT), lambda i: (0, 0)),
        pl.BlockSpec((1, _OUT), lambda i: (0, 0)),
    ],
    out_specs=pl.BlockSpec((_G, _OUT), lambda i: (0, 0)),
    out_shape=jax.ShapeDtypeStruct((_G, _OUT), jnp.float32),
)


def kernel(x, edge_index, edge_attr, batch, W_enc, b_enc, eps_all,
           W_edge_all, b_edge_all, W1_all, b1_all, W2_all, b2_all,
           W_fc, b_fc):
    src = edge_index[0].astype(jnp.int32)
    dst = edge_index[1].astype(jnp.int32)
    src_r = src.reshape(_NT, _CH // _IG, _IG, _B)
    dst_r = dst.reshape(_NT, _CH // _IG, _IG, _B)
    batch_r = batch.astype(jnp.int32).reshape(_N // _NB, 1, _NB)

    h = _enc_call(x, W_enc, b_enc.reshape(1, _D))
    es = [_edge_call(edge_attr, W_edge_all[i], b_edge_all[i].reshape(1, _D))
          for i in range(_L)]
    for i in range(_L):
        agg = _sc_agg(h, es[i], src_r, dst_r)
        h = _mlp_call(h, agg, (1.0 + eps_all[i]).reshape(1, 1),
                      W1_all[i], b1_all[i].reshape(1, _D),
                      W2_all[i], b2_all[i].reshape(1, _D))
    return _pool_call(h, batch_r, W_fc, b_fc.reshape(1, _OUT))
